# baseline (speedup 1.0000x reference)
import jax
import jax.numpy as jnp
from jax import lax
from jax.experimental import pallas as pl
from jax.experimental.pallas import tpu as pltpu
from jax.experimental.pallas import tpu_sc as plsc

NUM_CLASS = 1000
N_CLS_CTX = 4
CTX_DIM = 512
PREFIX_LEN = 6
SUFFIX_LEN = 67
SEQ_LEN = PREFIX_LEN + N_CLS_CTX + SUFFIX_LEN
BATCH = 1024
LANES = 16
NCOL = CTX_DIM // LANES

NC = 2
NS = 16
NW = NC * NS
BPW = BATCH // NW

N_BCAST = SEQ_LEN - N_CLS_CTX
UPS = 16
N_UNIT = N_BCAST * UPS
UROWS = BATCH // UPS
REP = 32
DMA_PER_UNIT = UROWS // REP
MAX_SLABS = 4
SUF_OFF = 8
STAGE_ROWS = SUF_OFF + SUFFIX_LEN
CCH = 8


def _body(cls_hbm, idx_hbm, pref_hbm, suf_hbm, out_hbm,
          idx_v, stage_v, rep_all, rows_v, crep0, crep1,
          gsem, bsem, bsem2, csem0, csem1, ssem):
    wid = lax.axis_index("s") * NC + lax.axis_index("c")
    base = wid * BPW

    c1 = pltpu.async_copy(idx_hbm.at[pl.ds(base, BPW)], idx_v, ssem)
    c2 = pltpu.async_copy(
        pref_hbm, stage_v.at[pl.ds(0, PREFIX_LEN * CTX_DIM)], ssem)
    c3 = pltpu.async_copy(
        suf_hbm, stage_v.at[pl.ds(SUF_OFF * CTX_DIM, SUFFIX_LEN * CTX_DIM)],
        ssem)
    c1.wait(); c2.wait(); c3.wait()

    start = (wid * N_UNIT) >> 5
    end = ((wid + 1) * N_UNIT) >> 5
    s_first = start >> 4
    bsems = [bsem, bsem2]

    def slab_bounds(k):
        lo = jnp.maximum(start, (s_first + k) * UPS)
        hi = jnp.minimum(end, (s_first + k + 1) * UPS)
        return lo, jnp.maximum(lo, hi)

    for k in range(MAX_SLABS):
        lo, hi = slab_bounds(k)

        @pl.when(lo < hi)
        def _slab(k=k, lo=lo, hi=hi):
            sem = bsems[k % 2]
            if k >= 2:
                plo, phi = slab_bounds(k - 2)

                def dr(i, c):
                    for q in range(DMA_PER_UNIT):
                        pltpu.make_async_copy(
                            out_hbm.at[0, pl.ds(0, REP)],
                            rep_all.at[0], sem).wait()
                    return c
                lax.fori_loop(0, phi - plo, dr, 0)

            ub = s_first + k
            sout = jnp.where(ub >= PREFIX_LEN, ub + N_CLS_CTX, ub)
            srcoff = jnp.where(ub >= PREFIX_LEN, ub + 2, ub) * CTX_DIM
            vals = [stage_v[pl.ds(srcoff + c * LANES, LANES)]
                    for c in range(NCOL)]

            def fr(row, vs):
                for c in range(NCOL):
                    rep_all[k % 2, row, pl.ds(c * LANES, LANES)] = vs[c]
                return vs
            lax.fori_loop(0, REP, fr, vals)

            def fire(u, c):
                row0 = (u & (UPS - 1)) * UROWS
                for q in range(DMA_PER_UNIT):
                    pltpu.async_copy(
                        rep_all.at[k % 2],
                        out_hbm.at[sout, pl.ds(row0 + q * REP, REP)], sem)
                return c
            lax.fori_loop(lo, hi, fire, 0)

    creps = [crep0, crep1]
    csems = [csem0, csem1]
    for ch in range(BPW // CCH):
        pltpu.async_copy(
            cls_hbm.at[idx_v.at[pl.ds(ch * CCH, CCH)]], rows_v, gsem).wait()
        for r in range(N_CLS_CTX):
            crep = creps[r % 2]
            csem = csems[r % 2]
            if ch > 0 or r >= 2:
                pltpu.make_async_copy(
                    out_hbm.at[0, pl.ds(0, CCH)], crep, csem).wait()

            def tpose(j, carry):
                for c in range(NCOL):
                    crep[j, pl.ds(c * LANES, LANES)] = (
                        rows_v[j, r, pl.ds(c * LANES, LANES)])
                return carry
            lax.fori_loop(0, CCH, tpose, 0)

            pltpu.async_copy(
                crep,
                out_hbm.at[PREFIX_LEN + r, pl.ds(base + ch * CCH, CCH)],
                csem)

    n1lo, n1hi = slab_bounds(1)
    n2lo, n2hi = slab_bounds(2)
    n3lo, n3hi = slab_bounds(3)
    rem_a = n2hi - n2lo
    n3 = n3hi - n3lo
    rem_b = jnp.where(n3 > 0, n3, n1hi - n1lo)

    def drain_a(i, c):
        for q in range(DMA_PER_UNIT):
            pltpu.make_async_copy(
                out_hbm.at[0, pl.ds(0, REP)], rep_all.at[0], bsem).wait()
        return c
    lax.fori_loop(0, rem_a, drain_a, 0)

    def drain_b(i, c):
        for q in range(DMA_PER_UNIT):
            pltpu.make_async_copy(
                out_hbm.at[0, pl.ds(0, REP)], rep_all.at[0], bsem2).wait()
        return c
    lax.fori_loop(0, rem_b, drain_b, 0)

    pltpu.make_async_copy(out_hbm.at[0, pl.ds(0, CCH)], crep0, csem0).wait()
    pltpu.make_async_copy(out_hbm.at[0, pl.ds(0, CCH)], crep1, csem1).wait()


@jax.jit
def _prompt_learner(label, cls_ctx, pref, suf):
    mesh = plsc.VectorSubcoreMesh(core_axis_name="c", subcore_axis_name="s")
    out_t = pl.kernel(
        _body,
        out_type=jax.ShapeDtypeStruct((SEQ_LEN, BATCH, CTX_DIM), jnp.float32),
        mesh=mesh,
        scratch_types=[
            pltpu.VMEM((BPW,), jnp.int32),
            pltpu.VMEM((STAGE_ROWS * CTX_DIM,), jnp.float32),
            pltpu.VMEM((2, REP, CTX_DIM), jnp.float32),
            pltpu.VMEM((CCH, N_CLS_CTX, CTX_DIM), jnp.float32),
            pltpu.VMEM((CCH, CTX_DIM), jnp.float32),
            pltpu.VMEM((CCH, CTX_DIM), jnp.float32),
            pltpu.SemaphoreType.DMA,
            pltpu.SemaphoreType.DMA,
            pltpu.SemaphoreType.DMA,
            pltpu.SemaphoreType.DMA,
            pltpu.SemaphoreType.DMA,
            pltpu.SemaphoreType.DMA,
        ],
    )(cls_ctx, label, pref, suf)
    return jnp.transpose(out_t, (1, 0, 2))


def kernel(label, cls_ctx, token_prefix, token_suffix):
    label = label.astype(jnp.int32)
    pref = token_prefix.reshape(PREFIX_LEN * CTX_DIM)
    suf = token_suffix.reshape(SUFFIX_LEN * CTX_DIM)
    return _prompt_learner(label, cls_ctx, pref, suf)


# speedup vs baseline: 1.1169x; 1.1169x over previous
import jax
import jax.numpy as jnp
from jax import lax
from jax.experimental import pallas as pl
from jax.experimental.pallas import tpu as pltpu
from jax.experimental.pallas import tpu_sc as plsc

NUM_CLASS = 1000
N_CLS_CTX = 4
CTX_DIM = 512
PREFIX_LEN = 6
SUFFIX_LEN = 67
SEQ_LEN = PREFIX_LEN + N_CLS_CTX + SUFFIX_LEN
BATCH = 1024
LANES = 16
NCOL = CTX_DIM // LANES

NC = 2
NS = 16
NW = NC * NS
BPW = BATCH // NW

N_BCAST = SEQ_LEN - N_CLS_CTX
UPS = 16
N_UNIT = N_BCAST * UPS
UROWS = BATCH // UPS
REP = 16
DMA_PER_UNIT = UROWS // REP
SUF_OFF = 8
STAGE_ROWS = SUF_OFF + SUFFIX_LEN
CCH = 8


def _body(cls_hbm, idx_hbm, pref_hbm, suf_hbm, out_hbm,
          idx_v, stage_v, rep_all, rows_v, crep0, crep1,
          gsem, bsem, csem0, csem1, ssem):
    wid = lax.axis_index("s") * NC + lax.axis_index("c")
    base = wid * BPW

    c1 = pltpu.async_copy(idx_hbm.at[pl.ds(base, BPW)], idx_v, ssem)
    c2 = pltpu.async_copy(
        pref_hbm, stage_v.at[pl.ds(0, PREFIX_LEN * CTX_DIM)], ssem)
    c3 = pltpu.async_copy(
        suf_hbm, stage_v.at[pl.ds(SUF_OFF * CTX_DIM, SUFFIX_LEN * CTX_DIM)],
        ssem)
    c1.wait(); c2.wait(); c3.wait()

    start = (wid * N_UNIT) >> 5
    end = ((wid + 1) * N_UNIT) >> 5
    s_first = start >> 4

    def unit_body(u, carry):
        ub = u >> 4
        bucket = ub - s_first
        sout = jnp.where(ub >= PREFIX_LEN, ub + N_CLS_CTX, ub)
        srcoff = jnp.where(ub >= PREFIX_LEN, ub + 2, ub) * CTX_DIM

        @pl.when((u == start) | ((u & (UPS - 1)) == 0))
        def _fill():
            vals = [stage_v[pl.ds(srcoff + c * LANES, LANES)]
                    for c in range(NCOL)]

            def fr(row, vs):
                for c in range(NCOL):
                    rep_all[bucket, row, pl.ds(c * LANES, LANES)] = vs[c]
                return vs
            lax.fori_loop(0, REP, fr, vals)

        row0 = (u & (UPS - 1)) * UROWS
        for q in range(DMA_PER_UNIT):
            pltpu.async_copy(
                rep_all.at[bucket],
                out_hbm.at[sout, pl.ds(row0 + q * REP, REP)], bsem)
        return carry

    lax.fori_loop(start, end, unit_body, 0)

    creps = [crep0, crep1]
    csems = [csem0, csem1]

    def cls_chunk(ch, carry):
        off = pl.multiple_of(ch * CCH, CCH)
        pltpu.async_copy(
            cls_hbm.at[idx_v.at[pl.ds(off, CCH)]], rows_v, gsem).wait()
        boff = pl.multiple_of(base + ch * CCH, CCH)
        for r in range(N_CLS_CTX):
            crep = creps[r % 2]
            csem = csems[r % 2]

            def _drain(crep=crep, csem=csem):
                pltpu.make_async_copy(
                    out_hbm.at[0, pl.ds(0, CCH)], crep, csem).wait()
            if r >= 2:
                _drain()
            else:
                pl.when(ch > 0)(_drain)

            def tpose(j, c2):
                for c in range(NCOL):
                    crep[j, pl.ds(c * LANES, LANES)] = (
                        rows_v[j, r, pl.ds(c * LANES, LANES)])
                return c2
            lax.fori_loop(0, CCH, tpose, 0)

            pltpu.async_copy(
                crep, out_hbm.at[PREFIX_LEN + r, pl.ds(boff, CCH)], csem)
        return carry

    lax.fori_loop(0, BPW // CCH, cls_chunk, 0)

    def drain(i, carry):
        for q in range(DMA_PER_UNIT):
            pltpu.make_async_copy(
                out_hbm.at[0, pl.ds(0, REP)], rep_all.at[0], bsem).wait()
        return carry
    lax.fori_loop(0, end - start, drain, 0)
    pltpu.make_async_copy(out_hbm.at[0, pl.ds(0, CCH)], crep0, csem0).wait()
    pltpu.make_async_copy(out_hbm.at[0, pl.ds(0, CCH)], crep1, csem1).wait()


@jax.jit
def _prompt_learner(label, cls_ctx, pref, suf):
    mesh = plsc.VectorSubcoreMesh(core_axis_name="c", subcore_axis_name="s")
    out_t = pl.kernel(
        _body,
        out_type=jax.ShapeDtypeStruct((SEQ_LEN, BATCH, CTX_DIM), jnp.float32),
        mesh=mesh,
        scratch_types=[
            pltpu.VMEM((BPW,), jnp.int32),
            pltpu.VMEM((STAGE_ROWS * CTX_DIM,), jnp.float32),
            pltpu.VMEM((4, REP, CTX_DIM), jnp.float32),
            pltpu.VMEM((CCH, N_CLS_CTX, CTX_DIM), jnp.float32),
            pltpu.VMEM((CCH, CTX_DIM), jnp.float32),
            pltpu.VMEM((CCH, CTX_DIM), jnp.float32),
            pltpu.SemaphoreType.DMA,
            pltpu.SemaphoreType.DMA,
            pltpu.SemaphoreType.DMA,
            pltpu.SemaphoreType.DMA,
            pltpu.SemaphoreType.DMA,
        ],
    )(cls_ctx, label, pref, suf)
    return jnp.transpose(out_t, (1, 0, 2))


def kernel(label, cls_ctx, token_prefix, token_suffix):
    label = label.astype(jnp.int32)
    pref = token_prefix.reshape(PREFIX_LEN * CTX_DIM)
    suf = token_suffix.reshape(SUFFIX_LEN * CTX_DIM)
    return _prompt_learner(label, cls_ctx, pref, suf)


# speedup vs baseline: 1.2753x; 1.1419x over previous
import jax
import jax.numpy as jnp
from jax import lax
from jax.experimental import pallas as pl
from jax.experimental.pallas import tpu as pltpu
from jax.experimental.pallas import tpu_sc as plsc

NUM_CLASS = 1000
N_CLS_CTX = 4
CTX_DIM = 512
PREFIX_LEN = 6
SUFFIX_LEN = 67
SEQ_LEN = PREFIX_LEN + N_CLS_CTX + SUFFIX_LEN
BATCH = 1024
LANES = 16
NCOL = CTX_DIM // LANES

NC = 2
NS = 16
NW = NC * NS
BPW = BATCH // NW

N_BCAST = SEQ_LEN - N_CLS_CTX
UPS = 16
N_UNIT = N_BCAST * UPS
UROWS = BATCH // UPS
REP = 32
DMA_PER_UNIT = UROWS // REP
STAGE = 8
CCH = 8


def _body(cls_hbm, idx_hbm, tmpl_hbm, out_hbm,
          idx_v, stage_v, rep_all, rows_v, crep0, crep1,
          gsem, bsem, csem0, csem1, ssem):
    wid = lax.axis_index("s") * NC + lax.axis_index("c")
    base = wid * BPW

    start = (wid * N_UNIT) >> 5
    end = ((wid + 1) * N_UNIT) >> 5
    s_first = start >> 4
    sout_first = jnp.where(s_first >= PREFIX_LEN,
                           s_first + N_CLS_CTX, s_first)

    c1 = pltpu.async_copy(idx_hbm.at[pl.ds(base, BPW)], idx_v, ssem)
    toff = pl.multiple_of(sout_first * CTX_DIM, CTX_DIM)
    c2 = pltpu.async_copy(
        tmpl_hbm.at[pl.ds(toff, STAGE * CTX_DIM)], stage_v, ssem)
    c1.wait(); c2.wait()

    def unit_body(u, carry):
        ub = u >> 4
        bucket = ub - s_first
        sout = jnp.where(ub >= PREFIX_LEN, ub + N_CLS_CTX, ub)
        srcoff = (sout - sout_first) * CTX_DIM

        @pl.when((u == start) | ((u & (UPS - 1)) == 0))
        def _fill():
            vals = [stage_v[pl.ds(srcoff + c * LANES, LANES)]
                    for c in range(NCOL)]

            def fr(row, vs):
                for c in range(NCOL):
                    rep_all[bucket, row, pl.ds(c * LANES, LANES)] = vs[c]
                return vs
            lax.fori_loop(0, REP, fr, vals)

        row0 = (u & (UPS - 1)) * UROWS
        for q in range(DMA_PER_UNIT):
            pltpu.async_copy(
                rep_all.at[bucket],
                out_hbm.at[sout, pl.ds(row0 + q * REP, REP)], bsem)
        return carry

    lax.fori_loop(start, end, unit_body, 0)

    creps = [crep0, crep1]
    csems = [csem0, csem1]

    def cls_chunk(ch, carry):
        off = pl.multiple_of(ch * CCH, CCH)
        pltpu.async_copy(
            cls_hbm.at[idx_v.at[pl.ds(off, CCH)]], rows_v, gsem).wait()
        boff = pl.multiple_of(base + ch * CCH, CCH)
        for r in range(N_CLS_CTX):
            crep = creps[r % 2]
            csem = csems[r % 2]

            def _drain(crep=crep, csem=csem):
                pltpu.make_async_copy(
                    out_hbm.at[0, pl.ds(0, CCH)], crep, csem).wait()
            if r >= 2:
                _drain()
            else:
                pl.when(ch > 0)(_drain)

            def tpose(j, c2):
                for c in range(NCOL):
                    crep[j, pl.ds(c * LANES, LANES)] = (
                        rows_v[j, r, pl.ds(c * LANES, LANES)])
                return c2
            lax.fori_loop(0, CCH, tpose, 0)

            pltpu.async_copy(
                crep, out_hbm.at[PREFIX_LEN + r, pl.ds(boff, CCH)], csem)
        return carry

    lax.fori_loop(0, BPW // CCH, cls_chunk, 0)

    def drain(i, carry):
        for q in range(DMA_PER_UNIT):
            pltpu.make_async_copy(
                out_hbm.at[0, pl.ds(0, REP)], rep_all.at[0], bsem).wait()
        return carry
    lax.fori_loop(0, end - start, drain, 0)
    pltpu.make_async_copy(out_hbm.at[0, pl.ds(0, CCH)], crep0, csem0).wait()
    pltpu.make_async_copy(out_hbm.at[0, pl.ds(0, CCH)], crep1, csem1).wait()


@jax.jit
def _prompt_learner(label, cls_ctx, tmpl):
    mesh = plsc.VectorSubcoreMesh(core_axis_name="c", subcore_axis_name="s")
    out_t = pl.kernel(
        _body,
        out_type=jax.ShapeDtypeStruct((SEQ_LEN, BATCH, CTX_DIM), jnp.float32),
        mesh=mesh,
        scratch_types=[
            pltpu.VMEM((BPW,), jnp.int32),
            pltpu.VMEM((STAGE * CTX_DIM,), jnp.float32),
            pltpu.VMEM((4, REP, CTX_DIM), jnp.float32),
            pltpu.VMEM((CCH, N_CLS_CTX, CTX_DIM), jnp.float32),
            pltpu.VMEM((CCH, CTX_DIM), jnp.float32),
            pltpu.VMEM((CCH, CTX_DIM), jnp.float32),
            pltpu.SemaphoreType.DMA,
            pltpu.SemaphoreType.DMA,
            pltpu.SemaphoreType.DMA,
            pltpu.SemaphoreType.DMA,
            pltpu.SemaphoreType.DMA,
        ],
    )(cls_ctx, label, tmpl)
    return jnp.transpose(out_t, (1, 0, 2))


def kernel(label, cls_ctx, token_prefix, token_suffix):
    label = label.astype(jnp.int32)
    tmpl = jnp.concatenate(
        [token_prefix.reshape(PREFIX_LEN * CTX_DIM),
         jnp.zeros((N_CLS_CTX * CTX_DIM,), jnp.float32),
         token_suffix.reshape(SUFFIX_LEN * CTX_DIM)])
    return _prompt_learner(label, cls_ctx, tmpl)
